# baseline (speedup 1.0000x reference)
import functools

import jax
import jax.numpy as jnp
from jax.experimental import pallas as pl
from jax.experimental.pallas import tpu as pltpu


def _moe_body(x_ref, f_ref, g_ref, lb_ref, wh_ref, wf_ref, cr_ref,
              w2b_ref, b2b_ref, w2i_ref, b2i_ref, w1_ref, b1_ref,
              we2_ref, be2_ref, alpha_ref,
              oh_ref, ogw_ref, ogl_ref, obw_ref, obl_ref, od_ref,
              *, NB, ES, DH):
    NE = NB * ES
    x = x_ref[...]
    f = f_ref[...]
    m = jnp.mean(x, axis=-1, keepdims=True)
    xc = x - m
    v = jnp.mean(xc * xc, axis=-1, keepdims=True)
    hn = xc * jax.lax.rsqrt(v + 1e-5) * g_ref[...] + lb_ref[...]
    hr = jax.nn.gelu(
        jnp.dot(hn, wh_ref[...], preferred_element_type=jnp.float32)
        + jnp.dot(f, wf_ref[...], preferred_element_type=jnp.float32)
        + cr_ref[...])
    bl = jnp.dot(hr, w2b_ref[...], preferred_element_type=jnp.float32) + b2b_ref[...]
    il = jnp.dot(hr, w2i_ref[...], preferred_element_type=jnp.float32) + b2i_ref[...]
    bm = jnp.max(bl, axis=-1, keepdims=True)
    be = jnp.exp(bl - bm)
    bw = be / jnp.sum(be, axis=-1, keepdims=True)
    im = jnp.max(il, axis=-1, keepdims=True)
    ie = jnp.exp(il - im)
    jj = jax.lax.broadcasted_iota(jnp.int32, (NE, NE), 0)
    kk = jax.lax.broadcasted_iota(jnp.int32, (NE, NE), 1)
    grp = (jj // ES == kk // ES).astype(jnp.float32)
    isum = jnp.dot(ie, grp, preferred_element_type=jnp.float32)
    iw = ie / isum
    bb = jax.lax.broadcasted_iota(jnp.int32, (NB, NE), 0)
    bk = jax.lax.broadcasted_iota(jnp.int32, (NB, NE), 1)
    rep = (bk // ES == bb).astype(jnp.float32)
    gw = jnp.dot(bw, rep, preferred_element_type=jnp.float32) * iw
    gl = jnp.dot(bl, rep, preferred_element_type=jnp.float32) + il
    h1 = jax.nn.gelu(
        jnp.dot(hn.astype(jnp.bfloat16), w1_ref[...],
                preferred_element_type=jnp.float32) + b1_ref[...])
    acc = jnp.dot(gw, be2_ref[...], preferred_element_type=jnp.float32)
    for k in range(NE):
        acc = acc + jnp.dot(
            (h1[:, DH * k: DH * (k + 1)] * gw[:, k:k + 1]).astype(jnp.bfloat16),
            we2_ref[k], preferred_element_type=jnp.float32)
    oh_ref[...] = x + alpha_ref[0, 0] * acc
    od_ref[...] = acc
    ogw_ref[...] = gw
    ogl_ref[...] = gl
    obw_ref[...] = bw
    obl_ref[...] = bl


def kernel(hidden, feat, ln_g, ln_b, Wsf, bsf, Wbf, bbf, Wbr1, bbr1, Wbr2, bbr2,
           Wir1, bir1, Wir2, bir2, We1, be1, We2, be2, alpha):
    B, T, D = hidden.shape
    F = feat.shape[-1]
    NB, FG, FE = Wbf.shape
    RH = Wbr1.shape[1]
    DH = We1.shape[2]
    ES = Wir2.shape[2]
    NE = NB * ES
    N = B * T
    RTOT = RH * (NB + 1)

    Wbr1_h, Wbr1_f = Wbr1[:D], Wbr1[D:]
    wh_parts = [Wbr1_h]
    wf = jnp.zeros((F, RTOT), jnp.float32)
    wf = wf.at[:, :RH].set(Wsf @ Wbr1_f)
    cr_parts = [bsf @ Wbr1_f + bbr1]
    for b in range(NB):
        Wi_h, Wi_f = Wir1[b][:D], Wir1[b][D:]
        wh_parts.append(Wi_h)
        wf = wf.at[FG * b:FG * (b + 1), RH * (b + 1):RH * (b + 2)].set(Wbf[b] @ Wi_f)
        cr_parts.append(bbf[b] @ Wi_f + bir1[b])
    wh = jnp.concatenate(wh_parts, axis=1)
    cr = jnp.concatenate(cr_parts)[None, :]
    w2b = jnp.zeros((RTOT, NB), jnp.float32).at[:RH].set(Wbr2)
    b2b = bbr2[None, :]
    w2i = jnp.zeros((RTOT, NE), jnp.float32)
    for b in range(NB):
        w2i = w2i.at[RH * (b + 1):RH * (b + 2), ES * b:ES * (b + 1)].set(Wir2[b])
    b2i = bir2.reshape(-1)[None, :]
    w1 = We1.transpose(1, 0, 2).reshape(D, NE * DH).astype(jnp.bfloat16)
    b1 = be1.reshape(-1)[None, :]
    we2b = We2.astype(jnp.bfloat16)

    x = hidden.reshape(N, D)
    f2 = feat.reshape(N, F)
    g2 = ln_g[None, :]
    lb2 = ln_b[None, :]
    a2 = alpha.reshape(1, 1)

    BLK = 512 if N % 512 == 0 else N
    grid = (N // BLK,)

    def tok(i):
        return (i, 0)

    def fix(i):
        return (0, 0)

    out_shape = [
        jax.ShapeDtypeStruct((N, D), jnp.float32),
        jax.ShapeDtypeStruct((N, NE), jnp.float32),
        jax.ShapeDtypeStruct((N, NE), jnp.float32),
        jax.ShapeDtypeStruct((N, NB), jnp.float32),
        jax.ShapeDtypeStruct((N, NB), jnp.float32),
        jax.ShapeDtypeStruct((N, D), jnp.float32),
    ]
    outs = pl.pallas_call(
        functools.partial(_moe_body, NB=NB, ES=ES, DH=DH),
        grid=grid,
        in_specs=[
            pl.BlockSpec((BLK, D), tok),
            pl.BlockSpec((BLK, F), tok),
            pl.BlockSpec((1, D), fix),
            pl.BlockSpec((1, D), fix),
            pl.BlockSpec((D, RTOT), fix),
            pl.BlockSpec((F, RTOT), fix),
            pl.BlockSpec((1, RTOT), fix),
            pl.BlockSpec((RTOT, NB), fix),
            pl.BlockSpec((1, NB), fix),
            pl.BlockSpec((RTOT, NE), fix),
            pl.BlockSpec((1, NE), fix),
            pl.BlockSpec((D, NE * DH), fix),
            pl.BlockSpec((1, NE * DH), fix),
            pl.BlockSpec((NE, DH, D), lambda i: (0, 0, 0)),
            pl.BlockSpec((NE, D), fix),
            pl.BlockSpec((1, 1), fix),
        ],
        out_specs=[
            pl.BlockSpec((BLK, D), tok),
            pl.BlockSpec((BLK, NE), tok),
            pl.BlockSpec((BLK, NE), tok),
            pl.BlockSpec((BLK, NB), tok),
            pl.BlockSpec((BLK, NB), tok),
            pl.BlockSpec((BLK, D), tok),
        ],
        out_shape=out_shape,
        compiler_params=pltpu.CompilerParams(
            dimension_semantics=("arbitrary",),
        ),
    )(x, f2, g2, lb2, wh, wf, cr, w2b, b2b, w2i, b2i, w1, b1, we2b, be2, a2)

    nh, gw, gl, bw, bl, dl = outs
    return (nh.reshape(B, T, D), gw.reshape(B, T, NE), gl.reshape(B, T, NE),
            bw.reshape(B, T, NB), bl.reshape(B, T, NB), dl.reshape(B, T, D))


# speedup vs baseline: 1.0515x; 1.0515x over previous
import functools

import jax
import jax.numpy as jnp
from jax.experimental import pallas as pl
from jax.experimental.pallas import tpu as pltpu


def _moe_body(x_ref, f_ref, g_ref, lb_ref, wh_ref, wf_ref, cr_ref,
              w2b_ref, b2b_ref, w2i_ref, b2i_ref, we1_ref, be1_ref,
              we2_ref, be2_ref, alpha_ref,
              oh_ref, ogw_ref, ogl_ref, obw_ref, obl_ref, od_ref,
              *, NB, ES, DH):
    NE = NB * ES
    x = x_ref[...]
    f = f_ref[...]
    m = jnp.mean(x, axis=-1, keepdims=True)
    xc = x - m
    v = jnp.mean(xc * xc, axis=-1, keepdims=True)
    hn = xc * jax.lax.rsqrt(v + 1e-5) * g_ref[...] + lb_ref[...]
    hr = jax.nn.gelu(
        jnp.dot(hn, wh_ref[...], preferred_element_type=jnp.float32)
        + jnp.dot(f, wf_ref[...], preferred_element_type=jnp.float32)
        + cr_ref[...])
    bl = jnp.dot(hr, w2b_ref[...], preferred_element_type=jnp.float32) + b2b_ref[...]
    il = jnp.dot(hr, w2i_ref[...], preferred_element_type=jnp.float32) + b2i_ref[...]
    bm = jnp.max(bl, axis=-1, keepdims=True)
    be = jnp.exp(bl - bm)
    bw = be / jnp.sum(be, axis=-1, keepdims=True)
    im = jnp.max(il, axis=-1, keepdims=True)
    ie = jnp.exp(il - im)
    jj = jax.lax.broadcasted_iota(jnp.int32, (NE, NE), 0)
    kk = jax.lax.broadcasted_iota(jnp.int32, (NE, NE), 1)
    grp = (jj // ES == kk // ES).astype(jnp.float32)
    isum = jnp.dot(ie, grp, preferred_element_type=jnp.float32)
    iw = ie / isum
    bb = jax.lax.broadcasted_iota(jnp.int32, (NB, NE), 0)
    bk = jax.lax.broadcasted_iota(jnp.int32, (NB, NE), 1)
    rep = (bk // ES == bb).astype(jnp.float32)
    gw = jnp.dot(bw, rep, preferred_element_type=jnp.float32) * iw
    gl = jnp.dot(bl, rep, preferred_element_type=jnp.float32) + il
    acc = jnp.dot(gw, be2_ref[...], preferred_element_type=jnp.float32)
    for k in range(NE):
        h1k = jax.nn.gelu(
            jnp.dot(hn, we1_ref[k], preferred_element_type=jnp.float32)
            + be1_ref[k:k + 1, :])
        acc = acc + jnp.dot(h1k * gw[:, k:k + 1], we2_ref[k],
                            preferred_element_type=jnp.float32)
    oh_ref[...] = x + alpha_ref[0, 0] * acc
    od_ref[...] = acc
    ogw_ref[...] = gw
    ogl_ref[...] = gl
    obw_ref[...] = bw
    obl_ref[...] = bl


def kernel(hidden, feat, ln_g, ln_b, Wsf, bsf, Wbf, bbf, Wbr1, bbr1, Wbr2, bbr2,
           Wir1, bir1, Wir2, bir2, We1, be1, We2, be2, alpha):
    B, T, D = hidden.shape
    F = feat.shape[-1]
    NB, FG, FE = Wbf.shape
    RH = Wbr1.shape[1]
    DH = We1.shape[2]
    ES = Wir2.shape[2]
    NE = NB * ES
    N = B * T
    RTOT = RH * (NB + 1)

    Wbr1_h, Wbr1_f = Wbr1[:D], Wbr1[D:]
    wh_parts = [Wbr1_h]
    wf = jnp.zeros((F, RTOT), jnp.float32)
    wf = wf.at[:, :RH].set(Wsf @ Wbr1_f)
    cr_parts = [bsf @ Wbr1_f + bbr1]
    for b in range(NB):
        Wi_h, Wi_f = Wir1[b][:D], Wir1[b][D:]
        wh_parts.append(Wi_h)
        wf = wf.at[FG * b:FG * (b + 1), RH * (b + 1):RH * (b + 2)].set(Wbf[b] @ Wi_f)
        cr_parts.append(bbf[b] @ Wi_f + bir1[b])
    wh = jnp.concatenate(wh_parts, axis=1)
    cr = jnp.concatenate(cr_parts)[None, :]
    w2b = jnp.zeros((RTOT, NB), jnp.float32).at[:RH].set(Wbr2)
    b2b = bbr2[None, :]
    w2i = jnp.zeros((RTOT, NE), jnp.float32)
    for b in range(NB):
        w2i = w2i.at[RH * (b + 1):RH * (b + 2), ES * b:ES * (b + 1)].set(Wir2[b])
    b2i = bir2.reshape(-1)[None, :]

    x = hidden.reshape(N, D)
    f2 = feat.reshape(N, F)
    g2 = ln_g[None, :]
    lb2 = ln_b[None, :]
    a2 = alpha.reshape(1, 1)

    BLK = 512 if N % 512 == 0 else N
    grid = (N // BLK,)

    def tok(i):
        return (i, 0)

    def fix(i):
        return (0, 0)

    out_shape = [
        jax.ShapeDtypeStruct((N, D), jnp.float32),
        jax.ShapeDtypeStruct((N, NE), jnp.float32),
        jax.ShapeDtypeStruct((N, NE), jnp.float32),
        jax.ShapeDtypeStruct((N, NB), jnp.float32),
        jax.ShapeDtypeStruct((N, NB), jnp.float32),
        jax.ShapeDtypeStruct((N, D), jnp.float32),
    ]
    outs = pl.pallas_call(
        functools.partial(_moe_body, NB=NB, ES=ES, DH=DH),
        grid=grid,
        in_specs=[
            pl.BlockSpec((BLK, D), tok),
            pl.BlockSpec((BLK, F), tok),
            pl.BlockSpec((1, D), fix),
            pl.BlockSpec((1, D), fix),
            pl.BlockSpec((D, RTOT), fix),
            pl.BlockSpec((F, RTOT), fix),
            pl.BlockSpec((1, RTOT), fix),
            pl.BlockSpec((RTOT, NB), fix),
            pl.BlockSpec((1, NB), fix),
            pl.BlockSpec((RTOT, NE), fix),
            pl.BlockSpec((1, NE), fix),
            pl.BlockSpec((NE, D, DH), lambda i: (0, 0, 0)),
            pl.BlockSpec((NE, DH), fix),
            pl.BlockSpec((NE, DH, D), lambda i: (0, 0, 0)),
            pl.BlockSpec((NE, D), fix),
            pl.BlockSpec((1, 1), fix),
        ],
        out_specs=[
            pl.BlockSpec((BLK, D), tok),
            pl.BlockSpec((BLK, NE), tok),
            pl.BlockSpec((BLK, NE), tok),
            pl.BlockSpec((BLK, NB), tok),
            pl.BlockSpec((BLK, NB), tok),
            pl.BlockSpec((BLK, D), tok),
        ],
        out_shape=out_shape,
        compiler_params=pltpu.CompilerParams(
            dimension_semantics=("arbitrary",),
        ),
    )(x, f2, g2, lb2, wh, wf, cr, w2b, b2b, w2i, b2i, We1, be1, We2, be2, a2)

    nh, gw, gl, bw, bl, dl = outs
    return (nh.reshape(B, T, D), gw.reshape(B, T, NE), gl.reshape(B, T, NE),
            bw.reshape(B, T, NB), bl.reshape(B, T, NB), dl.reshape(B, T, D))
